# Initial kernel scaffold; baseline (speedup 1.0000x reference)
import jax
import jax.numpy as jnp
from jax.experimental import pallas as pl


def kernel(utterance_input, response_input, utterance_graph_adj, response_graph_adj, emb, conv_W, conv_b, cross_w, attention_coef, assign_weight, mp_W1, mp_b1, mp_W2, mp_b2, ffn_W1, ffn_b1, ffn_W2, ffn_b2):
    raise NotImplementedError("write your pallas kernel here")


# speedup vs baseline: 5.5556x; 5.5556x over previous
import functools

import jax
import jax.numpy as jnp
from jax import lax
from jax.experimental import pallas as pl
from jax.experimental.pallas import tpu as pltpu
from jax.experimental.pallas import tpu_sc as plsc

_B = 256
_D = 128
_NC = 2
_NS = 16
_E = 256
_N = 64


@functools.cache
def _make_sc_gather():
    mesh = plsc.VectorSubcoreMesh(core_axis_name="c", subcore_axis_name="s")

    @functools.partial(
        pl.kernel,
        mesh=mesh,
        out_type=jax.ShapeDtypeStruct((_B, _D), jnp.float32),
        scratch_types=[
            pltpu.VMEM((8,), jnp.int32),
            pltpu.VMEM((8, _D), jnp.float32),
            pltpu.SemaphoreType.DMA,
        ],
    )
    def gather_k(table_hbm, idx_hbm, out_hbm, idx_v, rows_v, sem):
        wid = lax.axis_index("s") * _NC + lax.axis_index("c")
        base = wid * 8
        pltpu.sync_copy(idx_hbm.at[pl.ds(base, 8)], idx_v)
        pltpu.async_copy(table_hbm.at[idx_v], rows_v, sem).wait()
        pltpu.sync_copy(rows_v, out_hbm.at[pl.ds(base, 8)])

    return gather_k


def _gather_rows(emb, idx):
    return _make_sc_gather()(emb, idx)


def _compute_body(g_ref, urow_ref, ucol_ref, rrow_ref, rcol_ref,
                  convW_ref, convb_ref, crossw_ref, att_ref, aw_ref,
                  W1a_ref, W1b_ref, b1_ref, W2_ref, b2_ref,
                  F1u_ref, F1r_ref, F1m_ref, F1d_ref, fb1_ref, F2_ref, fb2_ref,
                  out_ref):
    f32 = jnp.float32
    g = g_ref[...]
    u = g[0:_N]
    r = g[_N:2 * _N]
    convW = convW_ref[...]
    convb = convb_ref[...]

    def gcn(x, row2d, col_row):
        R = (row2d == lax.broadcasted_iota(jnp.int32, (_E, _N), 1)).astype(f32)
        CT = (lax.broadcasted_iota(jnp.int32, (_N, _E), 0) == col_row).astype(f32)
        A = jnp.dot(CT, R, preferred_element_type=f32)
        deg = jnp.sum(A, axis=1, keepdims=True)
        dis = jnp.where(deg > 0, lax.rsqrt(deg), 0.0)
        xw = jnp.dot(x, convW, preferred_element_type=f32)
        out = dis * jnp.dot(A, dis * xw, preferred_element_type=f32)
        return out + convb

    u_self = gcn(u, urow_ref[...], ucol_ref[...])[0:50]
    r_self = gcn(r, rrow_ref[...], rcol_ref[...])[0:50]

    crossw = crossw_ref[...]
    uw = jnp.dot(u, crossw, preferred_element_type=f32)[0:50]
    rw = jnp.dot(r, crossw, preferred_element_type=f32)[0:50]
    att = att_ref[...]
    u_cross = jnp.dot(att, rw, preferred_element_type=f32)
    r_cross = jnp.dot(att, uw, preferred_element_type=f32)

    aw = aw_ref[...]
    W1a = W1a_ref[...]
    W1b = W1b_ref[...]
    b1 = b1_ref[...]
    W2 = W2_ref[...]
    b2 = b2_ref[...]

    def mpm(self_f, cross_f):
        a = aw * self_f
        b = aw * cross_f
        dot = jnp.sum(a * b, axis=1, keepdims=True)
        na = jnp.sqrt(jnp.sum(a * a, axis=1, keepdims=True))
        nb = jnp.sqrt(jnp.sum(b * b, axis=1, keepdims=True))
        dist = dot / (jnp.maximum(na, 1e-8) * jnp.maximum(nb, 1e-8))
        h = jnp.dot(self_f, W1b, preferred_element_type=f32) + dist * W1a + b1
        h = jnp.maximum(h, 0.0)
        return jnp.dot(h, W2, preferred_element_type=f32) + b2

    gu = jnp.max(mpm(u_self, u_cross), axis=0, keepdims=True)
    gr = jnp.max(mpm(r_self, r_cross), axis=0, keepdims=True)

    feat_h = (jnp.dot(gu, F1u_ref[...], preferred_element_type=f32)
              + jnp.dot(gr, F1r_ref[...], preferred_element_type=f32)
              + jnp.dot(gu * gr, F1m_ref[...], preferred_element_type=f32)
              + jnp.dot(jnp.abs(gu - gr), F1d_ref[...], preferred_element_type=f32)
              + fb1_ref[...])
    feat_h = jnp.maximum(feat_h, 0.0)
    logits = jnp.dot(feat_h, F2_ref[...], preferred_element_type=f32) + fb2_ref[...]
    out_ref[...] = 1.0 / (1.0 + jnp.exp(-logits))


def _compute_call(gathered, urow, ucol, rrow, rcol, conv_W, conv_b, cross_w,
                  attention_coef, assign_weight, mp_W1, mp_b1, mp_W2, mp_b2,
                  ffn_W1, ffn_b1, ffn_W2, ffn_b2):
    return pl.pallas_call(
        _compute_body,
        out_shape=jax.ShapeDtypeStruct((1, 1), jnp.float32),
    )(gathered, urow, ucol, rrow, rcol,
      conv_W, conv_b.reshape(1, 10), cross_w, attention_coef, assign_weight,
      mp_W1[0:1], mp_W1[1:11], mp_b1.reshape(1, 20), mp_W2, mp_b2.reshape(1, 20),
      ffn_W1[0:20], ffn_W1[20:40], ffn_W1[40:60], ffn_W1[60:80],
      ffn_b1.reshape(1, 40), ffn_W2, ffn_b2.reshape(1, 1))


def kernel(utterance_input, response_input, utterance_graph_adj, response_graph_adj,
           emb, conv_W, conv_b, cross_w, attention_coef, assign_weight,
           mp_W1, mp_b1, mp_W2, mp_b2, ffn_W1, ffn_b1, ffn_W2, ffn_b2):
    ui = utterance_input.astype(jnp.int32)
    ri = response_input.astype(jnp.int32)
    z14 = jnp.zeros((14,), jnp.int32)
    z142 = jnp.zeros((142,), jnp.int32)
    idx = jnp.concatenate([ui, z14, ri, z142])
    gathered = _gather_rows(emb, idx)

    loop = jnp.arange(50, dtype=jnp.int32)
    pad6 = jnp.full((6,), _N - 1, jnp.int32)

    def rc(adj):
        a = adj.astype(jnp.int32)
        row = jnp.concatenate([a[0], loop, pad6]).reshape(_E, 1)
        col = jnp.concatenate([a[1], loop, pad6]).reshape(1, _E)
        return row, col

    urow, ucol = rc(utterance_graph_adj)
    rrow, rcol = rc(response_graph_adj)

    out = _compute_call(gathered, urow, ucol, rrow, rcol, conv_W, conv_b,
                        cross_w, attention_coef, assign_weight, mp_W1, mp_b1,
                        mp_W2, mp_b2, ffn_W1, ffn_b1, ffn_W2, ffn_b2)
    return out.reshape(1)


# speedup vs baseline: 5.8686x; 1.0563x over previous
import functools

import jax
import jax.numpy as jnp
from jax import lax
from jax.experimental import pallas as pl
from jax.experimental.pallas import tpu as pltpu
from jax.experimental.pallas import tpu_sc as plsc

_B = 128
_D = 128
_NC = 2
_NS = 16
_E = 256
_N = 64


@functools.cache
def _make_sc_gather():
    mesh = plsc.VectorSubcoreMesh(core_axis_name="c", subcore_axis_name="s",
                                  num_cores=1)

    @functools.partial(
        pl.kernel,
        mesh=mesh,
        out_type=jax.ShapeDtypeStruct((_B, _D), jnp.float32),
        scratch_types=[
            pltpu.VMEM((8,), jnp.int32),
            pltpu.VMEM((8, _D), jnp.float32),
            pltpu.SemaphoreType.DMA,
        ],
    )
    def gather_k(table_hbm, idx_hbm, out_hbm, idx_v, rows_v, sem):
        wid = lax.axis_index("s")
        base = wid * 8
        pltpu.sync_copy(idx_hbm.at[pl.ds(base, 8)], idx_v)
        pltpu.async_copy(table_hbm.at[idx_v], rows_v, sem).wait()
        pltpu.sync_copy(rows_v, out_hbm.at[pl.ds(base, 8)])

    return gather_k


def _gather_rows(emb, idx):
    return _make_sc_gather()(emb, idx)


def _compute_body(g_ref, urow_ref, ucol_ref, rrow_ref, rcol_ref,
                  convW_ref, convb_ref, crossw_ref, att_ref, aw_ref,
                  W1a_ref, W1b_ref, b1_ref, W2_ref, b2_ref,
                  F1u_ref, F1r_ref, F1m_ref, F1d_ref, fb1_ref, F2_ref, fb2_ref,
                  out_ref):
    f32 = jnp.float32
    g = g_ref[...]
    u = g[0:_N]
    r = g[_N:2 * _N]
    convW = convW_ref[...]
    convb = convb_ref[...]

    def gcn(x, row2d, col_row):
        R = (row2d == lax.broadcasted_iota(jnp.int32, (_E, _N), 1)).astype(f32)
        CT = (lax.broadcasted_iota(jnp.int32, (_N, _E), 0) == col_row).astype(f32)
        A = jnp.dot(CT, R, preferred_element_type=f32)
        deg = jnp.sum(A, axis=1, keepdims=True)
        dis = jnp.where(deg > 0, lax.rsqrt(deg), 0.0)
        xw = jnp.dot(x, convW, preferred_element_type=f32)
        out = dis * jnp.dot(A, dis * xw, preferred_element_type=f32)
        return out + convb

    u_self = gcn(u, urow_ref[...], ucol_ref[...])[0:50]
    r_self = gcn(r, rrow_ref[...], rcol_ref[...])[0:50]

    crossw = crossw_ref[...]
    uw = jnp.dot(u, crossw, preferred_element_type=f32)[0:50]
    rw = jnp.dot(r, crossw, preferred_element_type=f32)[0:50]
    att = att_ref[...]
    u_cross = jnp.dot(att, rw, preferred_element_type=f32)
    r_cross = jnp.dot(att, uw, preferred_element_type=f32)

    aw = aw_ref[...]
    W1a = W1a_ref[...]
    W1b = W1b_ref[...]
    b1 = b1_ref[...]
    W2 = W2_ref[...]
    b2 = b2_ref[...]

    def mpm(self_f, cross_f):
        a = aw * self_f
        b = aw * cross_f
        dot = jnp.sum(a * b, axis=1, keepdims=True)
        na = jnp.sqrt(jnp.sum(a * a, axis=1, keepdims=True))
        nb = jnp.sqrt(jnp.sum(b * b, axis=1, keepdims=True))
        dist = dot / (jnp.maximum(na, 1e-8) * jnp.maximum(nb, 1e-8))
        h = jnp.dot(self_f, W1b, preferred_element_type=f32) + dist * W1a + b1
        h = jnp.maximum(h, 0.0)
        return jnp.dot(h, W2, preferred_element_type=f32) + b2

    gu = jnp.max(mpm(u_self, u_cross), axis=0, keepdims=True)
    gr = jnp.max(mpm(r_self, r_cross), axis=0, keepdims=True)

    feat_h = (jnp.dot(gu, F1u_ref[...], preferred_element_type=f32)
              + jnp.dot(gr, F1r_ref[...], preferred_element_type=f32)
              + jnp.dot(gu * gr, F1m_ref[...], preferred_element_type=f32)
              + jnp.dot(jnp.abs(gu - gr), F1d_ref[...], preferred_element_type=f32)
              + fb1_ref[...])
    feat_h = jnp.maximum(feat_h, 0.0)
    logits = jnp.dot(feat_h, F2_ref[...], preferred_element_type=f32) + fb2_ref[...]
    out_ref[...] = 1.0 / (1.0 + jnp.exp(-logits))


def _compute_call(gathered, urow, ucol, rrow, rcol, conv_W, conv_b, cross_w,
                  attention_coef, assign_weight, mp_W1, mp_b1, mp_W2, mp_b2,
                  ffn_W1, ffn_b1, ffn_W2, ffn_b2):
    return pl.pallas_call(
        _compute_body,
        out_shape=jax.ShapeDtypeStruct((1, 1), jnp.float32),
    )(gathered, urow, ucol, rrow, rcol,
      conv_W, conv_b.reshape(1, 10), cross_w, attention_coef, assign_weight,
      mp_W1[0:1], mp_W1[1:11], mp_b1.reshape(1, 20), mp_W2, mp_b2.reshape(1, 20),
      ffn_W1[0:20], ffn_W1[20:40], ffn_W1[40:60], ffn_W1[60:80],
      ffn_b1.reshape(1, 40), ffn_W2, ffn_b2.reshape(1, 1))


def kernel(utterance_input, response_input, utterance_graph_adj, response_graph_adj,
           emb, conv_W, conv_b, cross_w, attention_coef, assign_weight,
           mp_W1, mp_b1, mp_W2, mp_b2, ffn_W1, ffn_b1, ffn_W2, ffn_b2):
    ui = utterance_input.astype(jnp.int32)
    ri = response_input.astype(jnp.int32)
    z14 = jnp.zeros((14,), jnp.int32)
    idx = jnp.concatenate([ui, z14, ri, z14])
    gathered = _gather_rows(emb, idx)

    loop = jnp.arange(50, dtype=jnp.int32)
    pad6 = jnp.full((6,), _N - 1, jnp.int32)

    def rc(adj):
        a = adj.astype(jnp.int32)
        row = jnp.concatenate([a[0], loop, pad6]).reshape(_E, 1)
        col = jnp.concatenate([a[1], loop, pad6]).reshape(1, _E)
        return row, col

    urow, ucol = rc(utterance_graph_adj)
    rrow, rcol = rc(response_graph_adj)

    out = _compute_call(gathered, urow, ucol, rrow, rcol, conv_W, conv_b,
                        cross_w, attention_coef, assign_weight, mp_W1, mp_b1,
                        mp_W2, mp_b2, ffn_W1, ffn_b1, ffn_W2, ffn_b2)
    return out.reshape(1)


# speedup vs baseline: 7.4785x; 1.2743x over previous
import functools

import jax
import jax.numpy as jnp
from jax import lax
from jax.experimental import pallas as pl
from jax.experimental.pallas import tpu as pltpu
from jax.experimental.pallas import tpu_sc as plsc

_B = 128
_D = 128
_E = 200
_N = 64
_RO = 56


@functools.cache
def _make_sc_gather():
    mesh = plsc.VectorSubcoreMesh(core_axis_name="c", subcore_axis_name="s",
                                  num_cores=1)

    @functools.partial(
        pl.kernel,
        mesh=mesh,
        out_type=jax.ShapeDtypeStruct((_B, _D), jnp.float32),
        scratch_types=[
            pltpu.VMEM((8,), jnp.int32),
            pltpu.VMEM((8, _D), jnp.float32),
            pltpu.SemaphoreType.DMA,
        ],
    )
    def gather_k(table_hbm, utt_hbm, resp_hbm, out_hbm, idx_v, rows_v, sem):
        w = lax.axis_index("s")

        @pl.when(w < 6)
        def _():
            pltpu.sync_copy(utt_hbm.at[pl.ds(w * 8, 8)], idx_v)

        @pl.when(w == 6)
        def _():
            pltpu.sync_copy(utt_hbm.at[pl.ds(40, 8)], idx_v)
            pltpu.sync_copy(utt_hbm.at[pl.ds(48, 2)], idx_v.at[pl.ds(0, 2)])

        @pl.when((w >= 7) & (w < 13))
        def _():
            pltpu.sync_copy(resp_hbm.at[pl.ds((w - 7) * 8, 8)], idx_v)

        @pl.when(w == 13)
        def _():
            pltpu.sync_copy(resp_hbm.at[pl.ds(40, 8)], idx_v)
            pltpu.sync_copy(resp_hbm.at[pl.ds(48, 2)], idx_v.at[pl.ds(0, 2)])

        @pl.when(w >= 14)
        def _():
            pltpu.sync_copy(resp_hbm.at[pl.ds(0, 8)], idx_v)

        pltpu.async_copy(table_hbm.at[idx_v], rows_v, sem).wait()
        pltpu.sync_copy(rows_v, out_hbm.at[pl.ds(w * 8, 8)])

    return gather_k


def _gather_rows(emb, utt, resp):
    return _make_sc_gather()(emb, utt, resp)


def _compute_body(g_ref, uadj_ref, radj_ref,
                  convW_ref, convb_ref, crossw_ref, att_ref, aw_ref,
                  W1_ref, b1_ref, W2_ref, b2_ref,
                  F1_ref, fb1_ref, F2_ref, fb2_ref,
                  out_ref):
    f32 = jnp.float32
    i32 = jnp.int32
    g = g_ref[...]
    u = g[0:_N]
    r = g[_RO:_RO + _N]
    convW = convW_ref[...]
    convb = convb_ref[...]

    ii = lax.broadcasted_iota(i32, (_N, _N), 0)
    jj = lax.broadcasted_iota(i32, (_N, _N), 1)
    eye50 = jnp.where((ii == jj) & (ii < 50), 1.0, 0.0)
    e_iota = lax.broadcasted_iota(i32, (_N, _E), 0)

    def gcn(x, adj):
        rowv = adj[0:1, :]
        colv = adj[1:2, :]
        RT = (e_iota == rowv).astype(f32)
        CT = (e_iota == colv).astype(f32)
        A = lax.dot_general(CT, RT, (((1,), (1,)), ((), ())),
                            preferred_element_type=f32) + eye50
        deg = jnp.sum(A, axis=1, keepdims=True)
        dis = jnp.where(deg > 0, lax.rsqrt(deg), 0.0)
        xw = jnp.dot(x, convW, preferred_element_type=f32)
        return dis * jnp.dot(A, dis * xw, preferred_element_type=f32) + convb

    u_self = gcn(u, uadj_ref[...])[0:50]
    r_self = gcn(r, radj_ref[...])[0:50]

    crossw = crossw_ref[...]
    uw = jnp.dot(u, crossw, preferred_element_type=f32)[0:50]
    rw = jnp.dot(r, crossw, preferred_element_type=f32)[0:50]
    att = att_ref[...]
    u_cross = jnp.dot(att, rw, preferred_element_type=f32)
    r_cross = jnp.dot(att, uw, preferred_element_type=f32)

    aw = aw_ref[...]
    W1 = W1_ref[...]
    W1a = W1[0:1]
    W1b = W1[1:11]
    b1 = b1_ref[...]
    W2 = W2_ref[...]
    b2 = b2_ref[...]

    def mpm(self_f, cross_f):
        a = aw * self_f
        b = aw * cross_f
        dot = jnp.sum(a * b, axis=1, keepdims=True)
        na = jnp.sqrt(jnp.sum(a * a, axis=1, keepdims=True))
        nb = jnp.sqrt(jnp.sum(b * b, axis=1, keepdims=True))
        dist = dot / (jnp.maximum(na, 1e-8) * jnp.maximum(nb, 1e-8))
        h = jnp.dot(self_f, W1b, preferred_element_type=f32) + dist * W1a + b1
        h = jnp.maximum(h, 0.0)
        return jnp.dot(h, W2, preferred_element_type=f32) + b2

    gu = jnp.max(mpm(u_self, u_cross), axis=0, keepdims=True)
    gr = jnp.max(mpm(r_self, r_cross), axis=0, keepdims=True)

    F1 = F1_ref[...]
    feat_h = (jnp.dot(gu, F1[0:20], preferred_element_type=f32)
              + jnp.dot(gr, F1[20:40], preferred_element_type=f32)
              + jnp.dot(gu * gr, F1[40:60], preferred_element_type=f32)
              + jnp.dot(jnp.abs(gu - gr), F1[60:80], preferred_element_type=f32)
              + fb1_ref[...])
    feat_h = jnp.maximum(feat_h, 0.0)
    logits = jnp.dot(feat_h, F2_ref[...], preferred_element_type=f32) + fb2_ref[...]
    out_ref[...] = 1.0 / (1.0 + jnp.exp(-logits))


def kernel(utterance_input, response_input, utterance_graph_adj, response_graph_adj,
           emb, conv_W, conv_b, cross_w, attention_coef, assign_weight,
           mp_W1, mp_b1, mp_W2, mp_b2, ffn_W1, ffn_b1, ffn_W2, ffn_b2):
    ui = utterance_input.astype(jnp.int32)
    ri = response_input.astype(jnp.int32)
    gathered = _gather_rows(emb, ui, ri)

    out = pl.pallas_call(
        _compute_body,
        out_shape=jax.ShapeDtypeStruct((1, 1), jnp.float32),
    )(gathered,
      utterance_graph_adj.astype(jnp.int32), response_graph_adj.astype(jnp.int32),
      conv_W, conv_b.reshape(1, 10), cross_w, attention_coef, assign_weight,
      mp_W1, mp_b1.reshape(1, 20), mp_W2, mp_b2.reshape(1, 20),
      ffn_W1, ffn_b1.reshape(1, 40), ffn_W2, ffn_b2.reshape(1, 1))
    return out.reshape(1)


# speedup vs baseline: 7.5032x; 1.0033x over previous
import functools

import jax
import jax.numpy as jnp
from jax import lax
from jax.experimental import pallas as pl
from jax.experimental.pallas import tpu as pltpu
from jax.experimental.pallas import tpu_sc as plsc

_B = 128
_D = 128
_E = 200
_N = 64
_RO = 56


@functools.cache
def _make_sc_gather():
    mesh = plsc.VectorSubcoreMesh(core_axis_name="c", subcore_axis_name="s",
                                  num_cores=1)

    @functools.partial(
        pl.kernel,
        mesh=mesh,
        out_type=jax.ShapeDtypeStruct((_B, _D), jnp.float32),
        scratch_types=[
            pltpu.VMEM((8,), jnp.int32),
            pltpu.VMEM((8, _D), jnp.float32),
            pltpu.SemaphoreType.DMA,
        ],
    )
    def gather_k(table_hbm, utt_hbm, resp_hbm, out_hbm, idx_v, rows_v, sem):
        w = lax.axis_index("s")

        @pl.when(w < 6)
        def _():
            pltpu.sync_copy(utt_hbm.at[pl.ds(w * 8, 8)], idx_v)

        @pl.when(w == 6)
        def _():
            pltpu.sync_copy(utt_hbm.at[pl.ds(40, 8)], idx_v)
            pltpu.sync_copy(utt_hbm.at[pl.ds(48, 2)], idx_v.at[pl.ds(0, 2)])

        @pl.when((w >= 7) & (w < 13))
        def _():
            pltpu.sync_copy(resp_hbm.at[pl.ds((w - 7) * 8, 8)], idx_v)

        @pl.when(w == 13)
        def _():
            pltpu.sync_copy(resp_hbm.at[pl.ds(40, 8)], idx_v)
            pltpu.sync_copy(resp_hbm.at[pl.ds(48, 2)], idx_v.at[pl.ds(0, 2)])

        @pl.when(w >= 14)
        def _():
            pltpu.sync_copy(resp_hbm.at[pl.ds(0, 8)], idx_v)

        pltpu.async_copy(table_hbm.at[idx_v], rows_v, sem).wait()
        pltpu.sync_copy(rows_v, out_hbm.at[pl.ds(w * 8, 8)])

    return gather_k


def _gather_rows(emb, utt, resp):
    return _make_sc_gather()(emb, utt, resp)


def _compute_body(g_ref, uadj_ref, radj_ref,
                  convW_ref, convb_ref, crossw_ref, att_ref, aw_ref,
                  W1_ref, b1_ref, W2_ref, b2_ref,
                  F1_ref, fb1_ref, F2_ref, fb2_ref,
                  out_ref):
    f32 = jnp.float32
    i32 = jnp.int32
    g = g_ref[...]
    u = g[0:_N]
    r = g[_RO:_RO + _N]
    convW = convW_ref[...]
    convb = convb_ref[...]

    ii = lax.broadcasted_iota(i32, (_N, _N), 0)
    jj = lax.broadcasted_iota(i32, (_N, _N), 1)
    eye50 = jnp.where((ii == jj) & (ii < 50), 1.0, 0.0)
    e_iota = lax.broadcasted_iota(i32, (_N, _E), 0)

    def gcn(x, adj):
        rowv = adj[0:1, :]
        colv = adj[1:2, :]
        RT = (e_iota == rowv).astype(f32)
        CT = (e_iota == colv).astype(f32)
        A = lax.dot_general(CT, RT, (((1,), (1,)), ((), ())),
                            preferred_element_type=f32) + eye50
        deg = jnp.sum(A, axis=1, keepdims=True)
        dis = jnp.where(deg > 0, lax.rsqrt(deg), 0.0)
        xw = jnp.dot(x, convW, preferred_element_type=f32)
        return dis * jnp.dot(A, dis * xw, preferred_element_type=f32) + convb

    u_self = gcn(u, uadj_ref[...])[0:50]
    r_self = gcn(r, radj_ref[...])[0:50]

    crossw = crossw_ref[...]
    uw = jnp.dot(u, crossw, preferred_element_type=f32)[0:50]
    rw = jnp.dot(r, crossw, preferred_element_type=f32)[0:50]
    att = att_ref[...]
    u_cross = jnp.dot(att, rw, preferred_element_type=f32)
    r_cross = jnp.dot(att, uw, preferred_element_type=f32)

    aw = aw_ref[...]
    W1 = W1_ref[...]
    W1a = W1[0:1]
    W1b = W1[1:11]
    b1 = b1_ref[...]
    W2 = W2_ref[...]
    b2 = b2_ref[...]

    def mpm(self_f, cross_f):
        a = aw * self_f
        b = aw * cross_f
        dot = jnp.sum(a * b, axis=1, keepdims=True)
        na = jnp.sqrt(jnp.sum(a * a, axis=1, keepdims=True))
        nb = jnp.sqrt(jnp.sum(b * b, axis=1, keepdims=True))
        dist = dot / (jnp.maximum(na, 1e-8) * jnp.maximum(nb, 1e-8))
        h = jnp.dot(self_f, W1b, preferred_element_type=f32) + dist * W1a + b1
        h = jnp.maximum(h, 0.0)
        return jnp.dot(h, W2, preferred_element_type=f32) + b2

    gu = jnp.max(mpm(u_self, u_cross), axis=0, keepdims=True)
    gr = jnp.max(mpm(r_self, r_cross), axis=0, keepdims=True)

    F1 = F1_ref[...]
    feat_h = (jnp.dot(gu, F1[0:20], preferred_element_type=f32)
              + jnp.dot(gr, F1[20:40], preferred_element_type=f32)
              + jnp.dot(gu * gr, F1[40:60], preferred_element_type=f32)
              + jnp.dot(jnp.abs(gu - gr), F1[60:80], preferred_element_type=f32)
              + fb1_ref[...])
    feat_h = jnp.maximum(feat_h, 0.0)
    logits = jnp.dot(feat_h, F2_ref[...], preferred_element_type=f32) + fb2_ref[...]
    out_ref[...] = 1.0 / (1.0 + jnp.exp(-logits))


def kernel(utterance_input, response_input, utterance_graph_adj, response_graph_adj,
           emb, conv_W, conv_b, cross_w, attention_coef, assign_weight,
           mp_W1, mp_b1, mp_W2, mp_b2, ffn_W1, ffn_b1, ffn_W2, ffn_b2):
    ui = utterance_input.astype(jnp.int32)
    ri = response_input.astype(jnp.int32)
    gathered = _gather_rows(emb, ui, ri)

    out = pl.pallas_call(
        _compute_body,
        out_shape=jax.ShapeDtypeStruct((1, 1), jnp.float32),
    )(gathered,
      utterance_graph_adj.astype(jnp.int32), response_graph_adj.astype(jnp.int32),
      conv_W, conv_b, cross_w, attention_coef, assign_weight,
      mp_W1, mp_b1, mp_W2, mp_b2,
      ffn_W1, ffn_b1, ffn_W2, ffn_b2)
    return out.reshape(1)
